# baseline (speedup 1.0000x reference)
import functools

import jax
import jax.numpy as jnp
from jax import lax
from jax.experimental import pallas as pl
from jax.experimental.pallas import tpu as pltpu
from jax.experimental.pallas import tpu_sc as plsc

_L = 16
_NC, _NS = 2, 16

_MIN32 = -2147483648


def _matvec_kernel(x_ref, w_ref, o_ref):
    o_ref[...] = jnp.dot(x_ref[...], w_ref[...],
                         preferred_element_type=jnp.float32)


def _make_sc_mask(k, t, rows):
    n_work = _NC * _NS
    chunks = n_work // rows
    chunk = t // chunks
    nv_full = t // _L
    unroll = 16
    mesh = plsc.VectorSubcoreMesh(core_axis_name="c", subcore_axis_name="s")

    @functools.partial(
        pl.kernel, mesh=mesh,
        compiler_params=pltpu.CompilerParams(needs_layout_passes=False),
        out_type=jax.ShapeDtypeStruct((rows * t,), jnp.float32),
        scratch_types=[
            pltpu.VMEM((t,), jnp.float32),
            pltpu.VMEM((t,), jnp.int32),
            pltpu.VMEM((t + _L,), jnp.int32),
            pltpu.VMEM((t + _L,), jnp.int32),
            pltpu.VMEM((16 * _L,), jnp.int32),
            pltpu.VMEM((chunk,), jnp.float32),
        ],
    )
    def body(scores_hbm, out_hbm, sf, uk, ka, kb, h2, mv):
        cid = lax.axis_index("c")
        sid = lax.axis_index("s")
        wid = sid * _NC + cid
        row = wid // chunks
        ch = wid % chunks
        pltpu.sync_copy(scores_hbm.at[pl.ds(row * t, t)], sf)

        min32 = jnp.int32(_MIN32)
        one = jnp.int32(1)
        zero = jnp.int32(0)
        lane = lax.iota(jnp.int32, _L)
        zacc = jnp.zeros((_L,), jnp.int32)
        top_bit = jnp.int32(_MIN32)

        ones_v0 = jnp.full((_L,), 1, jnp.int32)
        for j in range(16):
            h2[pl.ds(j * _L, _L)] = zacc

        def tr(j8, carry):
            for jj in range(unroll):
                j = j8 * unroll + jj
                f = sf[pl.ds(j * _L, _L)]
                u = lax.bitcast_convert_type(f, jnp.int32)
                s = u ^ (lax.shift_right_arithmetic(u, 31) & jnp.int32(0x7FFFFFFF))
                ukv = s ^ min32
                uk[pl.ds(j * _L, _L)] = ukv
                dig = lax.shift_right_logical(ukv, 28)
                plsc.addupdate_scatter(h2, [lane * 16 + dig], ones_v0)
            return carry
        lax.fori_loop(0, nv_full // unroll, tr, 0)

        ones_v = jnp.full((_L,), 1, jnp.int32)

        def hist_pass(src_k, sh, m):
            for j in range(16):
                h2[pl.ds(j * _L, _L)] = zacc
            ng = (m + _L * unroll - 1) // (_L * unroll)

            def grp(j8, carry):
                for jj in range(unroll):
                    base = (j8 * unroll + jj) * _L
                    v = src_k[pl.ds(base, _L)]
                    valid = (lane + base) < m
                    dig = lax.shift_right_logical(v, sh) & jnp.int32(15)
                    plsc.addupdate_scatter(h2, [lane * 16 + dig], ones_v,
                                           mask=valid)
                return carry
            lax.fori_loop(0, ng, grp, 0)
            counts = zacc
            for j in range(16):
                counts = counts + h2[pl.ds(j * _L, _L)]
            return counts

        def level(src_k, dst_k, sh, m, cgt, p, do_compact, skip_hist=False):
            if skip_hist:
                counts = zacc
                for j in range(16):
                    counts = counts + h2[pl.ds(j * _L, _L)]
            else:
                counts = hist_pass(src_k, sh, m)
            ssum = lax.rev(plsc.cumsum(lax.rev(counts, (0,))), (0,))
            cond = (ssum + cgt) >= k
            beta = jnp.max(jnp.where(cond, lane, jnp.int32(-1)))
            bvec = jnp.full((_L,), 0, jnp.int32) + beta
            cntb = counts.at[bvec].get(mode="promise_in_bounds")[0]
            ssb = ssum.at[bvec].get(mode="promise_in_bounds")[0]
            cgt2 = cgt + (ssb - cntb)
            p2 = p | jnp.left_shift(beta, sh)
            if do_compact:
                ng = (m + _L * unroll - 1) // (_L * unroll)

                def grp(j8, run):
                    for jj in range(unroll):
                        base = (j8 * unroll + jj) * _L
                        v = src_k[pl.ds(base, _L)]
                        valid = (lane + base) < m
                        dig = lax.shift_right_logical(v, sh) & jnp.int32(15)
                        sel = (dig == beta) & valid
                        plsc.store_compressed(dst_k.at[pl.ds(run, _L)], v,
                                              mask=sel)
                        pc = plsc.all_reduce_population_count(sel)
                        run = run + pc[0]
                    return run
                lax.fori_loop(0, ng, grp, zero)
            return cntb, cgt2, p2

        m, cgt, p = jnp.int32(t), zero, zero
        bufs = [uk, kb, ka, kb, ka, kb, ka, kb, ka]
        for lvl in range(8):
            m, cgt, p = level(bufs[lvl], bufs[lvl + 1], 28 - 4 * lvl,
                              m, cgt, p, do_compact=(lvl < 7),
                              skip_hist=(lvl == 0))

        ks = p ^ min32
        need = k - cgt
        base = ch * chunk

        def prebody(j, acc):
            ukv = uk[pl.ds(j * _L, _L)]
            return acc + jnp.where(ukv == p, one, zero)
        prev = lax.fori_loop(0, base // _L, prebody, zacc)
        pre = jnp.sum(prev)

        def mb(j, run):
            ukv = uk[pl.ds(base + j * _L, _L)]
            eq = ukv == p
            gt = (ukv ^ min32) > ks
            eqi = jnp.where(eq, one, zero)
            incl = plsc.cumsum(eqi)
            rank = incl - eqi + run
            sel = gt | (eq & (rank < need))
            mv[pl.ds(j * _L, _L)] = jnp.where(sel, 1.0, 0.0).astype(jnp.float32)
            return run + incl[_L - 1]
        lax.fori_loop(0, chunk // _L, mb, pre)
        pltpu.sync_copy(mv, out_hbm.at[pl.ds(wid * chunk, chunk)])

    return body


def kernel(x, W):
    b, t, d = x.shape
    k = max(1, int(t * 0.5))
    x2 = x.reshape(b * t, d)
    tile = 1024
    grid = (b * t) // tile

    scores_col = pl.pallas_call(
        _matvec_kernel,
        grid=(grid,),
        in_specs=[
            pl.BlockSpec((tile, d), lambda i: (i, 0)),
            pl.BlockSpec((d, 1), lambda i: (0, 0)),
        ],
        out_specs=pl.BlockSpec((tile, 1), lambda i: (i, 0)),
        out_shape=jax.ShapeDtypeStruct((b * t, 1), jnp.float32),
    )(x2, W)

    mask_flat = _make_sc_mask(k, t, b)(scores_col.reshape(b * t))
    return (mask_flat.reshape(b, t, 1), scores_col.reshape(b, t))


# speedup vs baseline: 1.0479x; 1.0479x over previous
import functools

import jax
import jax.numpy as jnp
from jax import lax
from jax.experimental import pallas as pl
from jax.experimental.pallas import tpu as pltpu
from jax.experimental.pallas import tpu_sc as plsc

_L = 16
_NC, _NS = 2, 16

_MIN32 = -2147483648


def _matvec_kernel(x_ref, w_ref, o_ref):
    o_ref[...] = jnp.dot(x_ref[...], w_ref[...],
                         preferred_element_type=jnp.float32)


def _make_sc_mask(k, t, rows):
    n_work = _NC * _NS
    chunks = n_work // rows
    chunk = t // chunks
    nv_full = t // _L
    unroll = 8
    mesh = plsc.VectorSubcoreMesh(core_axis_name="c", subcore_axis_name="s")

    @functools.partial(
        pl.kernel, mesh=mesh,
        compiler_params=pltpu.CompilerParams(needs_layout_passes=False),
        out_type=jax.ShapeDtypeStruct((rows * t,), jnp.float32),
        scratch_types=[
            pltpu.VMEM((t,), jnp.float32),
            pltpu.VMEM((t,), jnp.int32),
            pltpu.VMEM((t + _L,), jnp.int32),
            pltpu.VMEM((t + _L,), jnp.int32),
            pltpu.VMEM((16 * _L,), jnp.int32),
            pltpu.VMEM((chunk,), jnp.float32),
        ],
    )
    def body(scores_hbm, out_hbm, sf, uk, ka, kb, h2, mv):
        cid = lax.axis_index("c")
        sid = lax.axis_index("s")
        wid = sid * _NC + cid
        row = wid // chunks
        ch = wid % chunks
        pltpu.sync_copy(scores_hbm.at[pl.ds(row * t, t)], sf)

        min32 = jnp.int32(_MIN32)
        one = jnp.int32(1)
        zero = jnp.int32(0)
        lane = lax.iota(jnp.int32, _L)
        zacc = jnp.zeros((_L,), jnp.int32)
        top_bit = jnp.int32(_MIN32)

        ones_v0 = jnp.full((_L,), 1, jnp.int32)
        for j in range(16):
            h2[pl.ds(j * _L, _L)] = zacc

        def tr(j8, carry):
            for jj in range(unroll):
                j = j8 * unroll + jj
                f = sf[pl.ds(j * _L, _L)]
                u = lax.bitcast_convert_type(f, jnp.int32)
                s = u ^ (lax.shift_right_arithmetic(u, 31) & jnp.int32(0x7FFFFFFF))
                ukv = s ^ min32
                uk[pl.ds(j * _L, _L)] = ukv
                dig = lax.shift_right_logical(ukv, 28)
                plsc.addupdate_scatter(h2, [lane * 16 + dig], ones_v0)
            return carry
        lax.fori_loop(0, nv_full // unroll, tr, 0)

        ones_v = jnp.full((_L,), 1, jnp.int32)

        def hist_pass(src_k, sh, m):
            for j in range(16):
                h2[pl.ds(j * _L, _L)] = zacc
            ng = (m + _L * unroll - 1) // (_L * unroll)

            def grp(j8, carry):
                for jj in range(unroll):
                    base = (j8 * unroll + jj) * _L
                    v = src_k[pl.ds(base, _L)]
                    valid = (lane + base) < m
                    dig = lax.shift_right_logical(v, sh) & jnp.int32(15)
                    plsc.addupdate_scatter(h2, [lane * 16 + dig], ones_v,
                                           mask=valid)
                return carry
            lax.fori_loop(0, ng, grp, 0)
            counts = zacc
            for j in range(16):
                counts = counts + h2[pl.ds(j * _L, _L)]
            return counts

        def level(src_k, dst_k, sh, m, cgt, p, do_compact, skip_hist=False):
            if skip_hist:
                counts = zacc
                for j in range(16):
                    counts = counts + h2[pl.ds(j * _L, _L)]
            else:
                counts = hist_pass(src_k, sh, m)
            ssum = lax.rev(plsc.cumsum(lax.rev(counts, (0,))), (0,))
            cond = (ssum + cgt) >= k
            beta = jnp.max(jnp.where(cond, lane, jnp.int32(-1)))
            bvec = jnp.full((_L,), 0, jnp.int32) + beta
            cntb = counts.at[bvec].get(mode="promise_in_bounds")[0]
            ssb = ssum.at[bvec].get(mode="promise_in_bounds")[0]
            cgt2 = cgt + (ssb - cntb)
            p2 = p | jnp.left_shift(beta, sh)
            if do_compact:
                ng = (m + _L * unroll - 1) // (_L * unroll)

                def grp(j8, run):
                    for jj in range(unroll):
                        base = (j8 * unroll + jj) * _L
                        v = src_k[pl.ds(base, _L)]
                        valid = (lane + base) < m
                        dig = lax.shift_right_logical(v, sh) & jnp.int32(15)
                        sel = (dig == beta) & valid
                        plsc.store_compressed(dst_k.at[pl.ds(run, _L)], v,
                                              mask=sel)
                        pc = plsc.all_reduce_population_count(sel)
                        run = run + pc[0]
                    return run
                lax.fori_loop(0, ng, grp, zero)
            return cntb, cgt2, p2

        m, cgt, p = jnp.int32(t), zero, zero
        bufs = [uk, kb, ka, kb, ka, kb, ka, kb, ka]
        for lvl in range(8):
            m, cgt, p = level(bufs[lvl], bufs[lvl + 1], 28 - 4 * lvl,
                              m, cgt, p, do_compact=(lvl < 7),
                              skip_hist=(lvl == 0))

        ks = p ^ min32
        need = k - cgt
        base = ch * chunk

        def prebody(j, acc):
            ukv = uk[pl.ds(j * _L, _L)]
            return acc + jnp.where(ukv == p, one, zero)
        prev = lax.fori_loop(0, base // _L, prebody, zacc)
        pre = jnp.sum(prev)

        def mb(j, run):
            ukv = uk[pl.ds(base + j * _L, _L)]
            eq = ukv == p
            gt = (ukv ^ min32) > ks
            eqi = jnp.where(eq, one, zero)
            incl = plsc.cumsum(eqi)
            rank = incl - eqi + run
            sel = gt | (eq & (rank < need))
            mv[pl.ds(j * _L, _L)] = jnp.where(sel, 1.0, 0.0).astype(jnp.float32)
            return run + incl[_L - 1]
        lax.fori_loop(0, chunk // _L, mb, pre)
        pltpu.sync_copy(mv, out_hbm.at[pl.ds(wid * chunk, chunk)])

    return body


def kernel(x, W):
    b, t, d = x.shape
    k = max(1, int(t * 0.5))
    x2 = x.reshape(b * t, d)
    tile = 1024
    grid = (b * t) // tile

    scores_col = pl.pallas_call(
        _matvec_kernel,
        grid=(grid,),
        in_specs=[
            pl.BlockSpec((tile, d), lambda i: (i, 0)),
            pl.BlockSpec((d, 1), lambda i: (0, 0)),
        ],
        out_specs=pl.BlockSpec((tile, 1), lambda i: (i, 0)),
        out_shape=jax.ShapeDtypeStruct((b * t, 1), jnp.float32),
    )(x2, W)

    mask_flat = _make_sc_mask(k, t, b)(scores_col.reshape(b * t))
    return (mask_flat.reshape(b, t, 1), scores_col.reshape(b, t))
